# Initial kernel scaffold; baseline (speedup 1.0000x reference)
import jax
import jax.numpy as jnp
from jax.experimental import pallas as pl


def kernel(input_ids, original_weight, new_weight):
    raise NotImplementedError("write your pallas kernel here")


# speedup vs baseline: 2.6541x; 2.6541x over previous
import functools

import jax
import jax.numpy as jnp
from jax import lax
from jax.experimental import pallas as pl
from jax.experimental.pallas import tpu as pltpu
from jax.experimental.pallas import tpu_sc as plsc

N_ORIG = 1000000
N_NEW = 1024
D = 64
B = 16384
L = 50

NC = 2
NS = 16
NW = NC * NS

TOKENS = B * L
TPW = TOKENS // NW
CHUNK = 512
NCHUNK = TPW // CHUNK
SUB = 128
NSUB = CHUNK // SUB


def _body(ids_hbm, orig_hbm, new_hbm, out_hbm, idxbuf, oidxbuf, rows,
          patchbuf, gsem):
    cid = lax.axis_index("c")
    sid = lax.axis_index("s")
    wid = sid * NC + cid
    base = wid * TPW

    @pl.loop(0, NCHUNK)
    def _chunk(c):
        cb = base + c * CHUNK
        pltpu.sync_copy(ids_hbm.at[pl.ds(cb, CHUNK)], idxbuf)

        @pl.loop(0, CHUNK // 16)
        def _grp(g):
            v = idxbuf[pl.ds(g * 16, 16)]
            ov = jnp.where(v >= N_ORIG, 0, v)
            oidxbuf[pl.ds(g * 16, 16)] = ov

        @pl.loop(0, NSUB)
        def _sub(s):
            pltpu.async_copy(
                orig_hbm.at[oidxbuf.at[pl.ds(s * SUB, SUB)]],
                rows.at[pl.ds(s * SUB, SUB)],
                gsem,
            ).wait()

        pltpu.sync_copy(rows, out_hbm.at[pl.ds(cb, CHUNK)])

        @pl.loop(0, CHUNK // 16)
        def _fix(g):
            v = idxbuf[pl.ds(g * 16, 16)]
            m = v >= N_ORIG
            cnt = plsc.all_reduce_population_count(m)[0]

            @pl.when(cnt > 0)
            def _():
                lane = lax.broadcasted_iota(jnp.int32, (16,), 0)
                nv = jnp.where(m, v - N_ORIG, 0)
                pltpu.async_copy(new_hbm.at[nv], patchbuf, gsem).wait()
                gdst = jnp.where(m, cb + g * 16 + lane, TOKENS + lane)
                pltpu.async_copy(patchbuf, out_hbm.at[gdst], gsem).wait()


@functools.partial(jax.jit, static_argnames=())
def kernel(input_ids, original_weight, new_weight):
    ids_flat = input_ids.reshape(TOKENS)
    mesh = plsc.VectorSubcoreMesh(
        core_axis_name="c", subcore_axis_name="s",
        num_cores=NC, num_subcores=NS,
    )
    out = pl.kernel(
        _body,
        out_type=jax.ShapeDtypeStruct((TOKENS + 16, D), jnp.float32),
        mesh=mesh,
        compiler_params=pltpu.CompilerParams(
            use_tc_tiling_on_sc=False, needs_layout_passes=False),
        scratch_types=[
            pltpu.VMEM((CHUNK,), jnp.int32),
            pltpu.VMEM((CHUNK,), jnp.int32),
            pltpu.VMEM((CHUNK, D), jnp.float32),
            pltpu.VMEM((16, D), jnp.float32),
            pltpu.SemaphoreType.DMA,
        ],
    )(ids_flat, original_weight, new_weight)
    return out[:TOKENS].reshape(B, L, D)
